# baseline (speedup 1.0000x reference)
import functools

import jax
import jax.numpy as jnp
from jax import lax
from jax.experimental import pallas as pl
from jax.experimental.pallas import tpu as pltpu
from jax.experimental.pallas import tpu_sc as plsc

BLK = 256


def _mm_body(be_ref, xs_ref, w_ref, b_ref, o_ref):
    acc = jnp.dot(xs_ref[...], w_ref[0], preferred_element_type=jnp.float32)
    o_ref[...] = jnp.maximum(acc + b_ref[0, 0][None, :], 0.0)


def _stacked_mm(xs, w, b, block_expert, nblocks):
    in_c = xs.shape[1]
    out_c = w.shape[2]
    grid_spec = pltpu.PrefetchScalarGridSpec(
        num_scalar_prefetch=1,
        grid=(nblocks,),
        in_specs=[
            pl.BlockSpec((BLK, in_c), lambda i, be: (i, 0)),
            pl.BlockSpec((1, in_c, out_c), lambda i, be: (be[i], 0, 0)),
            pl.BlockSpec((1, 1, out_c), lambda i, be: (be[i], 0, 0)),
        ],
        out_specs=pl.BlockSpec((BLK, out_c), lambda i, be: (i, 0)),
    )
    return pl.pallas_call(
        _mm_body,
        grid_spec=grid_spec,
        out_shape=jax.ShapeDtypeStruct((nblocks * BLK, out_c), jnp.float32),
    )(block_expert, xs, w, b)


def _sc_dispatch(x, pos_cols, npad):
    n_tok, c = x.shape
    k_top = len(pos_cols)
    info = plsc.get_sparse_core_info()
    nw = info.num_cores * info.num_subcores
    tpw = n_tok // nw
    tch = 32
    nch = tpw // tch
    p = jnp.stack(
        [pc.reshape(nw, nch, tch) for pc in pos_cols], axis=2
    )
    mesh = plsc.VectorSubcoreMesh(core_axis_name="c", subcore_axis_name="s")

    @functools.partial(
        pl.kernel,
        mesh=mesh,
        out_type=jax.ShapeDtypeStruct((npad, c), x.dtype),
        scratch_types=[
            pltpu.VMEM((nch, k_top, tch), jnp.int32),
            pltpu.VMEM((tch, c), x.dtype),
            pltpu.SemaphoreType.DMA,
        ],
    )
    def run(x_hbm, p_hbm, xs_hbm, idx_v, rows_v, sem):
        wid = lax.axis_index("s") * info.num_cores + lax.axis_index("c")
        base = wid * tpw
        pltpu.sync_copy(p_hbm.at[wid], idx_v)
        for ch in range(nch):
            pltpu.sync_copy(x_hbm.at[pl.ds(base + ch * tch, tch)], rows_v)
            copies = [
                pltpu.async_copy(rows_v, xs_hbm.at[idx_v.at[ch, kk]], sem)
                for kk in range(k_top)
            ]
            for cp in copies:
                cp.wait()

    return run(x, p)


def _sc_combine(ys, pos, n_rows):
    c = ys.shape[1]
    info = plsc.get_sparse_core_info()
    nw = info.num_cores * info.num_subcores
    rpw = n_rows // nw
    ch_sz = 32
    nch = rpw // ch_sz
    p = pos.reshape(nw, nch, ch_sz)
    mesh = plsc.VectorSubcoreMesh(core_axis_name="c", subcore_axis_name="s")

    @functools.partial(
        pl.kernel,
        mesh=mesh,
        out_type=jax.ShapeDtypeStruct((n_rows, c), jnp.float32),
        scratch_types=[
            pltpu.VMEM((nch, ch_sz), jnp.int32),
            pltpu.VMEM((ch_sz, c), jnp.float32),
            pltpu.SemaphoreType.DMA,
        ],
    )
    def run(ys_hbm, p_hbm, out_hbm, idx_v, rows_v, sem):
        wid = lax.axis_index("s") * info.num_cores + lax.axis_index("c")
        base = wid * rpw
        pltpu.sync_copy(p_hbm.at[wid], idx_v)
        for ch in range(nch):
            pltpu.async_copy(ys_hbm.at[idx_v.at[ch]], rows_v, sem).wait()
            pltpu.sync_copy(rows_v, out_hbm.at[pl.ds(base + ch * ch_sz, ch_sz)])

    return run(ys, p)


def kernel(x, idx, w, b):
    n_tok, in_c = x.shape
    k_top = idx.shape[1]
    n_exp = w.shape[0]
    out_c = w.shape[2]
    s = n_tok * k_top

    e = idx.reshape(s).astype(jnp.int32)
    oh = (e[:, None] == jnp.arange(n_exp, dtype=jnp.int32)[None, :]).astype(jnp.int32)
    csum = jnp.cumsum(oh, axis=0)
    cnt = csum[-1]
    rank = jnp.sum(csum * oh, axis=1) - 1
    blocks_e = (cnt + BLK - 1) // BLK
    cumblk = jnp.cumsum(blocks_e)
    nblocks = s // BLK + n_exp - 1
    start_row = (cumblk - blocks_e) * BLK
    pos = rank + jnp.sum(oh * start_row[None, :], axis=1)
    block_expert = jnp.minimum(
        jnp.sum(
            jnp.arange(nblocks, dtype=jnp.int32)[:, None] >= cumblk[None, :], axis=1
        ),
        n_exp - 1,
    ).astype(jnp.int32)

    pos2 = pos.reshape(n_tok, k_top)
    xb = lax.bitcast_convert_type(
        x.astype(jnp.bfloat16).reshape(n_tok, in_c // 2, 2), jnp.int32
    )
    wb = w.astype(jnp.bfloat16)
    xs32 = _sc_dispatch(xb, [pos2[:, kk] for kk in range(k_top)], nblocks * BLK)
    xs = lax.bitcast_convert_type(xs32, jnp.bfloat16).reshape(
        nblocks * BLK, in_c
    )
    ys = _stacked_mm(xs, wb, b, block_expert, nblocks)
    out_flat = _sc_combine(ys, pos, s)
    return out_flat.reshape(n_tok, k_top, out_c)


# speedup vs baseline: 2.2209x; 2.2209x over previous
import functools

import jax
import jax.numpy as jnp
from jax import lax
from jax.experimental import pallas as pl
from jax.experimental.pallas import tpu as pltpu
from jax.experimental.pallas import tpu_sc as plsc

BLK = 256


def _mm_body(be_ref, xs_ref, w_ref, b_ref, o_ref):
    acc = jnp.dot(xs_ref[...], w_ref[0], preferred_element_type=jnp.float32)
    o_ref[...] = jnp.maximum(acc + b_ref[0, 0][None, :], 0.0)


def _stacked_mm(xs, w, b, block_expert, nblocks):
    in_c = xs.shape[1]
    out_c = w.shape[2]
    grid_spec = pltpu.PrefetchScalarGridSpec(
        num_scalar_prefetch=1,
        grid=(nblocks,),
        in_specs=[
            pl.BlockSpec((BLK, in_c), lambda i, be: (i, 0)),
            pl.BlockSpec((1, in_c, out_c), lambda i, be: (be[i], 0, 0)),
            pl.BlockSpec((1, 1, out_c), lambda i, be: (be[i], 0, 0)),
        ],
        out_specs=pl.BlockSpec((BLK, out_c), lambda i, be: (i, 0)),
    )
    return pl.pallas_call(
        _mm_body,
        grid_spec=grid_spec,
        out_shape=jax.ShapeDtypeStruct((nblocks * BLK, out_c), jnp.float32),
    )(block_expert, xs, w, b)


def _sc_dispatch(x, pos_cols, npad):
    n_tok, c = x.shape
    k_top = len(pos_cols)
    info = plsc.get_sparse_core_info()
    nw = info.num_cores * info.num_subcores
    tpw = n_tok // nw
    tch = 32
    nch = tpw // tch
    p = jnp.stack(
        [pc.reshape(nw, nch, tch) for pc in pos_cols], axis=2
    )
    mesh = plsc.VectorSubcoreMesh(core_axis_name="c", subcore_axis_name="s")

    @functools.partial(
        pl.kernel,
        mesh=mesh,
        out_type=jax.ShapeDtypeStruct((npad, c), x.dtype),
        scratch_types=[
            pltpu.VMEM((nch, k_top, tch), jnp.int32),
            pltpu.VMEM((tch, c), x.dtype),
            pltpu.SemaphoreType.DMA,
        ],
    )
    def run(x_hbm, p_hbm, xs_hbm, idx_v, rows_v, sem):
        wid = lax.axis_index("s") * info.num_cores + lax.axis_index("c")
        base = wid * tpw
        pltpu.sync_copy(p_hbm.at[wid], idx_v)
        for ch in range(nch):
            pltpu.sync_copy(x_hbm.at[pl.ds(base + ch * tch, tch)], rows_v)
            copies = [
                pltpu.async_copy(rows_v, xs_hbm.at[idx_v.at[ch, kk]], sem)
                for kk in range(k_top)
            ]
            for cp in copies:
                cp.wait()

    return run(x, p)


def _sc_combine(ys, pos, n_rows):
    c = ys.shape[1]
    info = plsc.get_sparse_core_info()
    nw = info.num_cores * info.num_subcores
    rpw = n_rows // nw
    ch_sz = 32
    nch = rpw // ch_sz
    p = pos.reshape(nw, nch, ch_sz)
    mesh = plsc.VectorSubcoreMesh(core_axis_name="c", subcore_axis_name="s")

    @functools.partial(
        pl.kernel,
        mesh=mesh,
        out_type=jax.ShapeDtypeStruct((n_rows, c), jnp.float32),
        scratch_types=[
            pltpu.VMEM((nch, ch_sz), jnp.int32),
            pltpu.VMEM((ch_sz, c), jnp.float32),
            pltpu.SemaphoreType.DMA,
        ],
    )
    def run(ys_hbm, p_hbm, out_hbm, idx_v, rows_v, sem):
        wid = lax.axis_index("s") * info.num_cores + lax.axis_index("c")
        base = wid * rpw
        pltpu.sync_copy(p_hbm.at[wid], idx_v)
        for ch in range(nch):
            pltpu.async_copy(ys_hbm.at[idx_v.at[ch]], rows_v, sem).wait()
            pltpu.sync_copy(rows_v, out_hbm.at[pl.ds(base + ch * ch_sz, ch_sz)])

    return run(ys, p)


def kernel(x, idx, w, b):
    n_tok, in_c = x.shape
    k_top = idx.shape[1]
    n_exp = w.shape[0]
    out_c = w.shape[2]
    s = n_tok * k_top

    e = idx.reshape(s).astype(jnp.int32)
    oh = (e[:, None] == jnp.arange(n_exp, dtype=jnp.int32)[None, :]).astype(jnp.int32)
    csum = jnp.cumsum(oh, axis=0)
    cnt = csum[-1]
    rank = jnp.sum(csum * oh, axis=1) - 1
    blocks_e = (cnt + BLK - 1) // BLK
    cumblk = jnp.cumsum(blocks_e)
    nblocks = s // BLK + n_exp - 1
    start_row = (cumblk - blocks_e) * BLK
    pos = rank + jnp.sum(oh * start_row[None, :], axis=1)
    block_expert = jnp.minimum(
        jnp.sum(
            jnp.arange(nblocks, dtype=jnp.int32)[:, None] >= cumblk[None, :], axis=1
        ),
        n_exp - 1,
    ).astype(jnp.int32)

    pos2 = pos.reshape(n_tok, k_top)
    xs = _sc_dispatch(x, [pos2[:, kk] for kk in range(k_top)], nblocks * BLK)
    ys = _stacked_mm(
        xs.astype(jnp.bfloat16), w.astype(jnp.bfloat16), b, block_expert, nblocks
    )
    out_flat = _sc_combine(ys, pos, s)
    return out_flat.reshape(n_tok, k_top, out_c)


# speedup vs baseline: 2.5195x; 1.1344x over previous
import functools

import jax
import jax.numpy as jnp
from jax import lax
from jax.experimental import pallas as pl
from jax.experimental.pallas import tpu as pltpu
from jax.experimental.pallas import tpu_sc as plsc

BLK = 256


def _mm_body(be_ref, xs_ref, w_ref, b_ref, o_ref):
    e = be_ref[pl.program_id(0)]
    acc = jnp.dot(xs_ref[...], w_ref[e], preferred_element_type=jnp.float32)
    o_ref[...] = jnp.maximum(acc + b_ref[e, 0][None, :], 0.0)


def _stacked_mm(xs, w, b, block_expert, nblocks):
    n_exp, in_c, out_c = w.shape
    grid_spec = pltpu.PrefetchScalarGridSpec(
        num_scalar_prefetch=1,
        grid=(nblocks,),
        in_specs=[
            pl.BlockSpec((BLK, in_c), lambda i, be: (i, 0)),
            pl.BlockSpec((n_exp, in_c, out_c), lambda i, be: (0, 0, 0)),
            pl.BlockSpec((n_exp, 1, out_c), lambda i, be: (0, 0, 0)),
        ],
        out_specs=pl.BlockSpec((BLK, out_c), lambda i, be: (i, 0)),
    )
    return pl.pallas_call(
        _mm_body,
        grid_spec=grid_spec,
        out_shape=jax.ShapeDtypeStruct((nblocks * BLK, out_c), jnp.float32),
    )(block_expert, xs, w, b)


def _sc_dispatch(x, pos_cols, npad):
    n_tok, c = x.shape
    k_top = len(pos_cols)
    info = plsc.get_sparse_core_info()
    nw = info.num_cores * info.num_subcores
    tpw = n_tok // nw
    tch = 32
    nch = tpw // tch
    p = jnp.stack(
        [pc.reshape(nw, nch, tch) for pc in pos_cols], axis=2
    )
    mesh = plsc.VectorSubcoreMesh(core_axis_name="c", subcore_axis_name="s")

    @functools.partial(
        pl.kernel,
        mesh=mesh,
        out_type=jax.ShapeDtypeStruct((npad, c), x.dtype),
        scratch_types=[
            pltpu.VMEM((nch, k_top, tch), jnp.int32),
            pltpu.VMEM((tch, c), x.dtype),
            pltpu.SemaphoreType.DMA,
        ],
    )
    def run(x_hbm, p_hbm, xs_hbm, idx_v, rows_v, sem):
        wid = lax.axis_index("s") * info.num_cores + lax.axis_index("c")
        base = wid * tpw
        pltpu.sync_copy(p_hbm.at[wid], idx_v)
        for ch in range(nch):
            pltpu.sync_copy(x_hbm.at[pl.ds(base + ch * tch, tch)], rows_v)
            copies = [
                pltpu.async_copy(rows_v, xs_hbm.at[idx_v.at[ch, kk]], sem)
                for kk in range(k_top)
            ]
            for cp in copies:
                cp.wait()

    return run(x, p)


def _sc_combine(ys, pos, n_rows):
    c = ys.shape[1]
    info = plsc.get_sparse_core_info()
    nw = info.num_cores * info.num_subcores
    rpw = n_rows // nw
    ch_sz = 32
    nch = rpw // ch_sz
    p = pos.reshape(nw, nch, ch_sz)
    mesh = plsc.VectorSubcoreMesh(core_axis_name="c", subcore_axis_name="s")

    @functools.partial(
        pl.kernel,
        mesh=mesh,
        out_type=jax.ShapeDtypeStruct((n_rows, c), jnp.float32),
        scratch_types=[
            pltpu.VMEM((nch, ch_sz), jnp.int32),
            pltpu.VMEM((ch_sz, c), jnp.float32),
            pltpu.SemaphoreType.DMA,
        ],
    )
    def run(ys_hbm, p_hbm, out_hbm, idx_v, rows_v, sem):
        wid = lax.axis_index("s") * info.num_cores + lax.axis_index("c")
        base = wid * rpw
        pltpu.sync_copy(p_hbm.at[wid], idx_v)
        for ch in range(nch):
            pltpu.async_copy(ys_hbm.at[idx_v.at[ch]], rows_v, sem).wait()
            pltpu.sync_copy(rows_v, out_hbm.at[pl.ds(base + ch * ch_sz, ch_sz)])

    return run(ys, p)


def kernel(x, idx, w, b):
    n_tok, in_c = x.shape
    k_top = idx.shape[1]
    n_exp = w.shape[0]
    out_c = w.shape[2]
    s = n_tok * k_top

    e = idx.reshape(s).astype(jnp.int32)
    oh = (e[:, None] == jnp.arange(n_exp, dtype=jnp.int32)[None, :]).astype(jnp.int32)
    csum = jnp.cumsum(oh, axis=0)
    cnt = csum[-1]
    rank = jnp.sum(csum * oh, axis=1) - 1
    blocks_e = (cnt + BLK - 1) // BLK
    cumblk = jnp.cumsum(blocks_e)
    nblocks = s // BLK + n_exp - 1
    start_row = (cumblk - blocks_e) * BLK
    pos = rank + jnp.sum(oh * start_row[None, :], axis=1)
    block_expert = jnp.minimum(
        jnp.sum(
            jnp.arange(nblocks, dtype=jnp.int32)[:, None] >= cumblk[None, :], axis=1
        ),
        n_exp - 1,
    ).astype(jnp.int32)

    pos2 = pos.reshape(n_tok, k_top)
    xs = _sc_dispatch(x, [pos2[:, kk] for kk in range(k_top)], nblocks * BLK)
    ys = _stacked_mm(xs, w, b, block_expert, nblocks)
    out_flat = _sc_combine(ys, pos, s)
    return out_flat.reshape(n_tok, k_top, out_c)


# speedup vs baseline: 2.5964x; 1.0305x over previous
import functools

import jax
import jax.numpy as jnp
from jax import lax
from jax.experimental import pallas as pl
from jax.experimental.pallas import tpu as pltpu
from jax.experimental.pallas import tpu_sc as plsc

BLK = 512


def _mm_body(be_ref, xs_ref, w_ref, b_ref, o_ref):
    e = be_ref[pl.program_id(0)]
    acc = jnp.dot(xs_ref[...], w_ref[e], preferred_element_type=jnp.float32)
    o_ref[...] = jnp.maximum(acc + b_ref[e, 0][None, :], 0.0)


def _stacked_mm(xs, w, b, block_expert, nblocks):
    n_exp, in_c, out_c = w.shape
    grid_spec = pltpu.PrefetchScalarGridSpec(
        num_scalar_prefetch=1,
        grid=(nblocks,),
        in_specs=[
            pl.BlockSpec((BLK, in_c), lambda i, be: (i, 0)),
            pl.BlockSpec((n_exp, in_c, out_c), lambda i, be: (0, 0, 0)),
            pl.BlockSpec((n_exp, 1, out_c), lambda i, be: (0, 0, 0)),
        ],
        out_specs=pl.BlockSpec((BLK, out_c), lambda i, be: (i, 0)),
    )
    return pl.pallas_call(
        _mm_body,
        grid_spec=grid_spec,
        out_shape=jax.ShapeDtypeStruct((nblocks * BLK, out_c), jnp.float32),
    )(block_expert, xs, w, b)


def _sc_dispatch(x, pos_cols, npad):
    n_tok, c = x.shape
    k_top = len(pos_cols)
    info = plsc.get_sparse_core_info()
    nw = info.num_cores * info.num_subcores
    tpw = n_tok // nw
    tch = 32
    nch = tpw // tch
    p = jnp.stack(
        [pc.reshape(nw, nch, tch) for pc in pos_cols], axis=2
    )
    mesh = plsc.VectorSubcoreMesh(core_axis_name="c", subcore_axis_name="s")

    @functools.partial(
        pl.kernel,
        mesh=mesh,
        out_type=jax.ShapeDtypeStruct((npad, c), x.dtype),
        scratch_types=[
            pltpu.VMEM((nch, k_top, tch), jnp.int32),
            pltpu.VMEM((tch, c), x.dtype),
            pltpu.SemaphoreType.DMA,
        ],
    )
    def run(x_hbm, p_hbm, xs_hbm, idx_v, rows_v, sem):
        wid = lax.axis_index("s") * info.num_cores + lax.axis_index("c")
        base = wid * tpw
        pltpu.sync_copy(p_hbm.at[wid], idx_v)
        for ch in range(nch):
            pltpu.sync_copy(x_hbm.at[pl.ds(base + ch * tch, tch)], rows_v)
            copies = [
                pltpu.async_copy(rows_v, xs_hbm.at[idx_v.at[ch, kk]], sem)
                for kk in range(k_top)
            ]
            for cp in copies:
                cp.wait()

    return run(x, p)


def _sc_combine(ys, pos, n_rows):
    c = ys.shape[1]
    info = plsc.get_sparse_core_info()
    nw = info.num_cores * info.num_subcores
    rpw = n_rows // nw
    ch_sz = 32
    nch = rpw // ch_sz
    p = pos.reshape(nw, nch, ch_sz)
    mesh = plsc.VectorSubcoreMesh(core_axis_name="c", subcore_axis_name="s")

    @functools.partial(
        pl.kernel,
        mesh=mesh,
        out_type=jax.ShapeDtypeStruct((n_rows, c), jnp.float32),
        scratch_types=[
            pltpu.VMEM((nch, ch_sz), jnp.int32),
            pltpu.VMEM((ch_sz, c), jnp.float32),
            pltpu.SemaphoreType.DMA,
        ],
    )
    def run(ys_hbm, p_hbm, out_hbm, idx_v, rows_v, sem):
        wid = lax.axis_index("s") * info.num_cores + lax.axis_index("c")
        base = wid * rpw
        pltpu.sync_copy(p_hbm.at[wid], idx_v)
        for ch in range(nch):
            pltpu.async_copy(ys_hbm.at[idx_v.at[ch]], rows_v, sem).wait()
            pltpu.sync_copy(rows_v, out_hbm.at[pl.ds(base + ch * ch_sz, ch_sz)])

    return run(ys, p)


def kernel(x, idx, w, b):
    n_tok, in_c = x.shape
    k_top = idx.shape[1]
    n_exp = w.shape[0]
    out_c = w.shape[2]
    s = n_tok * k_top

    e = idx.reshape(s).astype(jnp.int32)
    oh = (e[:, None] == jnp.arange(n_exp, dtype=jnp.int32)[None, :]).astype(jnp.int32)
    csum = jnp.cumsum(oh, axis=0)
    cnt = csum[-1]
    rank = jnp.sum(csum * oh, axis=1) - 1
    blocks_e = (cnt + BLK - 1) // BLK
    cumblk = jnp.cumsum(blocks_e)
    nblocks = s // BLK + n_exp - 1
    start_row = (cumblk - blocks_e) * BLK
    pos = rank + jnp.sum(oh * start_row[None, :], axis=1)
    block_expert = jnp.minimum(
        jnp.sum(
            jnp.arange(nblocks, dtype=jnp.int32)[:, None] >= cumblk[None, :], axis=1
        ),
        n_exp - 1,
    ).astype(jnp.int32)

    pos2 = pos.reshape(n_tok, k_top)
    xs = _sc_dispatch(x, [pos2[:, kk] for kk in range(k_top)], nblocks * BLK)
    ys = _stacked_mm(xs, w, b, block_expert, nblocks)
    out_flat = _sc_combine(ys, pos, s)
    return out_flat.reshape(n_tok, k_top, out_c)


# speedup vs baseline: 2.6269x; 1.0118x over previous
import functools

import jax
import jax.numpy as jnp
from jax import lax
from jax.experimental import pallas as pl
from jax.experimental.pallas import tpu as pltpu
from jax.experimental.pallas import tpu_sc as plsc

BLK = 512


def _mm_body(be_ref, xs_ref, w_ref, b_ref, o_ref):
    e = be_ref[pl.program_id(0)]
    acc = jnp.dot(xs_ref[...], w_ref[e], preferred_element_type=jnp.float32)
    o_ref[...] = jnp.maximum(acc + b_ref[e, 0][None, :], 0.0)


def _stacked_mm(xs, w, b, block_expert, nblocks):
    n_exp, in_c, out_c = w.shape
    grid_spec = pltpu.PrefetchScalarGridSpec(
        num_scalar_prefetch=1,
        grid=(nblocks,),
        in_specs=[
            pl.BlockSpec((BLK, in_c), lambda i, be: (i, 0)),
            pl.BlockSpec((n_exp, in_c, out_c), lambda i, be: (0, 0, 0)),
            pl.BlockSpec((n_exp, 1, out_c), lambda i, be: (0, 0, 0)),
        ],
        out_specs=pl.BlockSpec((BLK, out_c), lambda i, be: (i, 0)),
    )
    return pl.pallas_call(
        _mm_body,
        grid_spec=grid_spec,
        out_shape=jax.ShapeDtypeStruct((nblocks * BLK, out_c), jnp.float32),
    )(block_expert, xs, w, b)


def _sc_dispatch(x, pos_cols, npad):
    n_tok, c = x.shape
    k_top = len(pos_cols)
    info = plsc.get_sparse_core_info()
    nw = info.num_cores * info.num_subcores
    tpw = n_tok // nw
    tch = 32
    nch = tpw // tch
    p = jnp.stack(
        [pc.reshape(nw, nch, tch) for pc in pos_cols], axis=2
    )
    mesh = plsc.VectorSubcoreMesh(core_axis_name="c", subcore_axis_name="s")

    @functools.partial(
        pl.kernel,
        mesh=mesh,
        out_type=jax.ShapeDtypeStruct((npad, c), x.dtype),
        scratch_types=[
            pltpu.VMEM((nch, k_top, tch), jnp.int32),
            pltpu.VMEM((2, tch, c), x.dtype),
            pltpu.SemaphoreType.DMA,
            pltpu.SemaphoreType.DMA,
        ],
    )
    def run(x_hbm, p_hbm, xs_hbm, idx_v, rows_v, rsem, ssem):
        wid = lax.axis_index("s") * info.num_cores + lax.axis_index("c")
        base = wid * tpw
        pltpu.sync_copy(p_hbm.at[wid], idx_v)
        pltpu.async_copy(x_hbm.at[pl.ds(base, tch)], rows_v.at[0], rsem).wait()
        prev = None
        for ch in range(nch):
            buf = ch % 2
            if ch + 1 < nch:
                if prev is not None:
                    for cp in prev:
                        cp.wait()
                rd = pltpu.async_copy(
                    x_hbm.at[pl.ds(base + (ch + 1) * tch, tch)],
                    rows_v.at[(ch + 1) % 2],
                    rsem,
                )
            cur = [
                pltpu.async_copy(rows_v.at[buf], xs_hbm.at[idx_v.at[ch, kk]], ssem)
                for kk in range(k_top)
            ]
            if ch + 1 < nch:
                rd.wait()
            prev = cur
        for cp in prev:
            cp.wait()

    return run(x, p)


def _sc_combine(ys, pos, n_rows):
    c = ys.shape[1]
    info = plsc.get_sparse_core_info()
    nw = info.num_cores * info.num_subcores
    rpw = n_rows // nw
    ch_sz = 32
    nch = rpw // ch_sz
    p = pos.reshape(nw, nch, ch_sz)
    mesh = plsc.VectorSubcoreMesh(core_axis_name="c", subcore_axis_name="s")

    @functools.partial(
        pl.kernel,
        mesh=mesh,
        out_type=jax.ShapeDtypeStruct((n_rows, c), jnp.float32),
        scratch_types=[
            pltpu.VMEM((nch, ch_sz), jnp.int32),
            pltpu.VMEM((2, ch_sz, c), jnp.float32),
            pltpu.SemaphoreType.DMA,
            pltpu.SemaphoreType.DMA,
        ],
    )
    def run(ys_hbm, p_hbm, out_hbm, idx_v, rows_v, gsem, wsem):
        wid = lax.axis_index("s") * info.num_cores + lax.axis_index("c")
        base = wid * rpw
        pltpu.sync_copy(p_hbm.at[wid], idx_v)
        gath = pltpu.async_copy(ys_hbm.at[idx_v.at[0]], rows_v.at[0], gsem)
        prev_w = None
        for ch in range(nch):
            buf = ch % 2
            gath.wait()
            if ch + 1 < nch:
                if prev_w is not None:
                    prev_w.wait()
                gath = pltpu.async_copy(
                    ys_hbm.at[idx_v.at[ch + 1]], rows_v.at[(ch + 1) % 2], gsem
                )
            prev_w = pltpu.async_copy(
                rows_v.at[buf], out_hbm.at[pl.ds(base + ch * ch_sz, ch_sz)], wsem
            )
        prev_w.wait()

    return run(ys, p)


def kernel(x, idx, w, b):
    n_tok, in_c = x.shape
    k_top = idx.shape[1]
    n_exp = w.shape[0]
    out_c = w.shape[2]
    s = n_tok * k_top

    e = idx.reshape(s).astype(jnp.int32)
    oh = (e[:, None] == jnp.arange(n_exp, dtype=jnp.int32)[None, :]).astype(jnp.int32)
    csum = jnp.cumsum(oh, axis=0)
    cnt = csum[-1]
    rank = jnp.sum(csum * oh, axis=1) - 1
    blocks_e = (cnt + BLK - 1) // BLK
    cumblk = jnp.cumsum(blocks_e)
    nblocks = s // BLK + n_exp - 1
    start_row = (cumblk - blocks_e) * BLK
    pos = rank + jnp.sum(oh * start_row[None, :], axis=1)
    block_expert = jnp.minimum(
        jnp.sum(
            jnp.arange(nblocks, dtype=jnp.int32)[:, None] >= cumblk[None, :], axis=1
        ),
        n_exp - 1,
    ).astype(jnp.int32)

    pos2 = pos.reshape(n_tok, k_top)
    xs = _sc_dispatch(x, [pos2[:, kk] for kk in range(k_top)], nblocks * BLK)
    ys = _stacked_mm(xs, w, b, block_expert, nblocks)
    out_flat = _sc_combine(ys, pos, s)
    return out_flat.reshape(n_tok, k_top, out_c)


# speedup vs baseline: 2.6509x; 1.0092x over previous
import functools

import jax
import jax.numpy as jnp
from jax import lax
from jax.experimental import pallas as pl
from jax.experimental.pallas import tpu as pltpu
from jax.experimental.pallas import tpu_sc as plsc

BLK = 512


def _mm_body(be_ref, xs_ref, w_ref, b_ref, o_ref):
    e = be_ref[pl.program_id(0)]
    acc = jnp.dot(xs_ref[...], w_ref[e], preferred_element_type=jnp.float32)
    o_ref[...] = jnp.maximum(acc + b_ref[e, 0][None, :], 0.0)


def _stacked_mm(xs, w, b, block_expert, nblocks):
    n_exp, in_c, out_c = w.shape
    grid_spec = pltpu.PrefetchScalarGridSpec(
        num_scalar_prefetch=1,
        grid=(nblocks,),
        in_specs=[
            pl.BlockSpec((BLK, in_c), lambda i, be: (i, 0)),
            pl.BlockSpec((n_exp, in_c, out_c), lambda i, be: (0, 0, 0)),
            pl.BlockSpec((n_exp, 1, out_c), lambda i, be: (0, 0, 0)),
        ],
        out_specs=pl.BlockSpec((BLK, out_c), lambda i, be: (i, 0)),
    )
    return pl.pallas_call(
        _mm_body,
        grid_spec=grid_spec,
        out_shape=jax.ShapeDtypeStruct((nblocks * BLK, out_c), jnp.float32),
    )(block_expert, xs, w, b)


def _sc_dispatch(x, pos_cols, npad):
    n_tok, c = x.shape
    k_top = len(pos_cols)
    info = plsc.get_sparse_core_info()
    nw = info.num_cores * info.num_subcores
    tpw = n_tok // nw
    tch = 32
    nch = tpw // tch
    p = jnp.stack(
        [pc.reshape(nw, nch, tch) for pc in pos_cols], axis=2
    )
    mesh = plsc.VectorSubcoreMesh(core_axis_name="c", subcore_axis_name="s")

    @functools.partial(
        pl.kernel,
        mesh=mesh,
        out_type=jax.ShapeDtypeStruct((npad, c), x.dtype),
        scratch_types=[
            pltpu.VMEM((nch, k_top, tch), jnp.int32),
            pltpu.VMEM((2, tch, c), x.dtype),
            pltpu.SemaphoreType.DMA,
            pltpu.SemaphoreType.DMA,
        ],
    )
    def run(x_hbm, p_hbm, xs_hbm, idx_v, rows_v, rsem, ssem):
        wid = lax.axis_index("s") * info.num_cores + lax.axis_index("c")
        base = wid * tpw
        pltpu.sync_copy(p_hbm.at[wid], idx_v)
        pltpu.async_copy(x_hbm.at[pl.ds(base, tch)], rows_v.at[0], rsem).wait()
        prev = None
        for ch in range(nch):
            buf = ch % 2
            if prev is not None:
                for cp in prev:
                    cp.wait()
                prev = None
            if ch + 1 < nch:
                rd = pltpu.async_copy(
                    x_hbm.at[pl.ds(base + (ch + 1) * tch, tch)],
                    rows_v.at[(ch + 1) % 2],
                    rsem,
                )
            cur = [
                pltpu.async_copy(rows_v.at[buf], xs_hbm.at[idx_v.at[ch, kk]], ssem)
                for kk in range(k_top)
            ]
            if ch + 1 < nch:
                rd.wait()
            prev = cur
        for cp in prev:
            cp.wait()

    return run(x, p)


def _sc_combine(ys, pos, n_rows):
    c = ys.shape[1]
    info = plsc.get_sparse_core_info()
    nw = info.num_cores * info.num_subcores
    rpw = n_rows // nw
    ch_sz = 32
    nch = rpw // ch_sz
    p = pos.reshape(nw, nch, ch_sz)
    mesh = plsc.VectorSubcoreMesh(core_axis_name="c", subcore_axis_name="s")

    @functools.partial(
        pl.kernel,
        mesh=mesh,
        out_type=jax.ShapeDtypeStruct((n_rows, c), jnp.float32),
        scratch_types=[
            pltpu.VMEM((nch, ch_sz), jnp.int32),
            pltpu.VMEM((2, ch_sz, c), jnp.float32),
            pltpu.SemaphoreType.DMA,
            pltpu.SemaphoreType.DMA,
        ],
    )
    def run(ys_hbm, p_hbm, out_hbm, idx_v, rows_v, gsem, wsem):
        wid = lax.axis_index("s") * info.num_cores + lax.axis_index("c")
        base = wid * rpw
        pltpu.sync_copy(p_hbm.at[wid], idx_v)
        gath = pltpu.async_copy(ys_hbm.at[idx_v.at[0]], rows_v.at[0], gsem)
        prev_w = None
        for ch in range(nch):
            buf = ch % 2
            gath.wait()
            if prev_w is not None:
                prev_w.wait()
                prev_w = None
            if ch + 1 < nch:
                gath = pltpu.async_copy(
                    ys_hbm.at[idx_v.at[ch + 1]], rows_v.at[(ch + 1) % 2], gsem
                )
            prev_w = pltpu.async_copy(
                rows_v.at[buf], out_hbm.at[pl.ds(base + ch * ch_sz, ch_sz)], wsem
            )
        prev_w.wait()

    return run(ys, p)


def kernel(x, idx, w, b):
    n_tok, in_c = x.shape
    k_top = idx.shape[1]
    n_exp = w.shape[0]
    out_c = w.shape[2]
    s = n_tok * k_top

    e = idx.reshape(s).astype(jnp.int32)
    oh = (e[:, None] == jnp.arange(n_exp, dtype=jnp.int32)[None, :]).astype(jnp.int32)
    csum = jnp.cumsum(oh, axis=0)
    cnt = csum[-1]
    rank = jnp.sum(csum * oh, axis=1) - 1
    blocks_e = (cnt + BLK - 1) // BLK
    cumblk = jnp.cumsum(blocks_e)
    nblocks = s // BLK + n_exp - 1
    start_row = (cumblk - blocks_e) * BLK
    pos = rank + jnp.sum(oh * start_row[None, :], axis=1)
    block_expert = jnp.minimum(
        jnp.sum(
            jnp.arange(nblocks, dtype=jnp.int32)[:, None] >= cumblk[None, :], axis=1
        ),
        n_exp - 1,
    ).astype(jnp.int32)

    pos2 = pos.reshape(n_tok, k_top)
    xs = _sc_dispatch(x, [pos2[:, kk] for kk in range(k_top)], nblocks * BLK)
    ys = _stacked_mm(xs, w, b, block_expert, nblocks)
    out_flat = _sc_combine(ys, pos, s)
    return out_flat.reshape(n_tok, k_top, out_c)


# speedup vs baseline: 2.6752x; 1.0092x over previous
import functools

import jax
import jax.numpy as jnp
from jax import lax
from jax.experimental import pallas as pl
from jax.experimental.pallas import tpu as pltpu
from jax.experimental.pallas import tpu_sc as plsc

BLK = 512


def _mm_body(be_ref, nb_ref, xs_ref, w_ref, b_ref, o_ref):
    i = pl.program_id(0)

    @pl.when(i < nb_ref[0])
    def _():
        e = be_ref[i]
        acc = jnp.dot(xs_ref[...], w_ref[e], preferred_element_type=jnp.float32)
        o_ref[...] = jnp.maximum(acc + b_ref[e, 0][None, :], 0.0)


def _stacked_mm(xs, w, b, block_expert, nb_used, nblocks):
    n_exp, in_c, out_c = w.shape
    grid_spec = pltpu.PrefetchScalarGridSpec(
        num_scalar_prefetch=2,
        grid=(nblocks,),
        in_specs=[
            pl.BlockSpec((BLK, in_c), lambda i, be, nb: (jnp.minimum(i, nb[0] - 1), 0)),
            pl.BlockSpec((n_exp, in_c, out_c), lambda i, be, nb: (0, 0, 0)),
            pl.BlockSpec((n_exp, 1, out_c), lambda i, be, nb: (0, 0, 0)),
        ],
        out_specs=pl.BlockSpec(
            (BLK, out_c), lambda i, be, nb: (jnp.minimum(i, nb[0] - 1), 0)
        ),
    )
    return pl.pallas_call(
        _mm_body,
        grid_spec=grid_spec,
        out_shape=jax.ShapeDtypeStruct((nblocks * BLK, out_c), jnp.float32),
    )(block_expert, nb_used, xs, w, b)


def _sc_dispatch(x, pos_cols, npad):
    n_tok, c = x.shape
    k_top = len(pos_cols)
    info = plsc.get_sparse_core_info()
    nw = info.num_cores * info.num_subcores
    tpw = n_tok // nw
    tch = 32
    nch = tpw // tch
    p = jnp.stack(
        [pc.reshape(nw, nch, tch) for pc in pos_cols], axis=2
    )
    mesh = plsc.VectorSubcoreMesh(core_axis_name="c", subcore_axis_name="s")

    @functools.partial(
        pl.kernel,
        mesh=mesh,
        out_type=jax.ShapeDtypeStruct((npad, c), x.dtype),
        scratch_types=[
            pltpu.VMEM((nch, k_top, tch), jnp.int32),
            pltpu.VMEM((tch, c), x.dtype),
            pltpu.SemaphoreType.DMA,
        ],
    )
    def run(x_hbm, p_hbm, xs_hbm, idx_v, rows_v, ssem):
        wid = lax.axis_index("s") * info.num_cores + lax.axis_index("c")
        base = wid * tpw
        pltpu.sync_copy(p_hbm.at[wid], idx_v)
        for ch in range(nch):
            pltpu.sync_copy(x_hbm.at[pl.ds(base + ch * tch, tch)], rows_v)
            copies = [
                pltpu.async_copy(rows_v, xs_hbm.at[idx_v.at[ch, kk]], ssem)
                for kk in range(k_top)
            ]
            for cp in copies:
                cp.wait()

    return run(x, p)


def _sc_combine(ys, pos, n_rows):
    c = ys.shape[1]
    info = plsc.get_sparse_core_info()
    nw = info.num_cores * info.num_subcores
    rpw = n_rows // nw
    ch_sz = 32
    nch = rpw // ch_sz
    p = pos.reshape(nw, nch, ch_sz)
    mesh = plsc.VectorSubcoreMesh(core_axis_name="c", subcore_axis_name="s")

    @functools.partial(
        pl.kernel,
        mesh=mesh,
        out_type=jax.ShapeDtypeStruct((n_rows, c), jnp.float32),
        scratch_types=[
            pltpu.VMEM((nch, ch_sz), jnp.int32),
            pltpu.VMEM((ch_sz, c), jnp.float32),
            pltpu.SemaphoreType.DMA,
        ],
    )
    def run(ys_hbm, p_hbm, out_hbm, idx_v, rows_v, gsem):
        wid = lax.axis_index("s") * info.num_cores + lax.axis_index("c")
        base = wid * rpw
        pltpu.sync_copy(p_hbm.at[wid], idx_v)
        for ch in range(nch):
            pltpu.async_copy(ys_hbm.at[idx_v.at[ch]], rows_v, gsem).wait()
            pltpu.sync_copy(rows_v, out_hbm.at[pl.ds(base + ch * ch_sz, ch_sz)])

    return run(ys, p)


def kernel(x, idx, w, b):
    n_tok, in_c = x.shape
    k_top = idx.shape[1]
    n_exp = w.shape[0]
    out_c = w.shape[2]
    s = n_tok * k_top

    e = idx.reshape(s).astype(jnp.int32)
    oh = (e[:, None] == jnp.arange(n_exp, dtype=jnp.int32)[None, :]).astype(jnp.int32)
    csum = jnp.cumsum(oh, axis=0)
    cnt = csum[-1]
    rank = jnp.sum(csum * oh, axis=1) - 1
    blocks_e = (cnt + BLK - 1) // BLK
    cumblk = jnp.cumsum(blocks_e)
    nblocks = s // BLK + n_exp - 1
    start_row = (cumblk - blocks_e) * BLK
    pos = rank + jnp.sum(oh * start_row[None, :], axis=1)
    block_expert = jnp.minimum(
        jnp.sum(
            jnp.arange(nblocks, dtype=jnp.int32)[:, None] >= cumblk[None, :], axis=1
        ),
        n_exp - 1,
    ).astype(jnp.int32)

    pos2 = pos.reshape(n_tok, k_top)
    nb_used = cumblk[-1:].astype(jnp.int32)
    xs = _sc_dispatch(x, [pos2[:, kk] for kk in range(k_top)], nblocks * BLK)
    ys = _stacked_mm(xs, w, b, block_expert, nb_used, nblocks)
    out_flat = _sc_combine(ys, pos, s)
    return out_flat.reshape(n_tok, k_top, out_c)


# speedup vs baseline: 2.7687x; 1.0350x over previous
import functools

import jax
import jax.numpy as jnp
from jax import lax
from jax.experimental import pallas as pl
from jax.experimental.pallas import tpu as pltpu
from jax.experimental.pallas import tpu_sc as plsc

BLK = 512


def _mm_body(be_ref, nb_ref, xs_ref, w_ref, b_ref, o_ref):
    i = pl.program_id(0)

    @pl.when(i < nb_ref[0])
    def _():
        e = be_ref[i]
        acc = jnp.dot(xs_ref[...], w_ref[e], preferred_element_type=jnp.float32)
        o_ref[...] = jnp.maximum(acc + b_ref[e, 0][None, :], 0.0)


def _stacked_mm(xs, w, b, block_expert, nb_used, nblocks):
    n_exp, in_c, out_c = w.shape
    grid_spec = pltpu.PrefetchScalarGridSpec(
        num_scalar_prefetch=2,
        grid=(nblocks,),
        in_specs=[
            pl.BlockSpec((BLK, in_c), lambda i, be, nb: (jnp.minimum(i, nb[0] - 1), 0)),
            pl.BlockSpec((n_exp, in_c, out_c), lambda i, be, nb: (0, 0, 0)),
            pl.BlockSpec((n_exp, 1, out_c), lambda i, be, nb: (0, 0, 0)),
        ],
        out_specs=pl.BlockSpec(
            (BLK, out_c), lambda i, be, nb: (jnp.minimum(i, nb[0] - 1), 0)
        ),
    )
    return pl.pallas_call(
        _mm_body,
        grid_spec=grid_spec,
        out_shape=jax.ShapeDtypeStruct((nblocks * BLK, out_c), jnp.float32),
    )(block_expert, nb_used, xs, w, b)


def _sc_dispatch(x, pos_cols, npad):
    n_tok, c = x.shape
    k_top = len(pos_cols)
    info = plsc.get_sparse_core_info()
    nw = info.num_cores * info.num_subcores
    tpw = n_tok // nw
    tch = 64
    nch = tpw // tch
    p = jnp.stack(
        [pc.reshape(nw, nch, tch) for pc in pos_cols], axis=2
    )
    mesh = plsc.VectorSubcoreMesh(core_axis_name="c", subcore_axis_name="s")

    @functools.partial(
        pl.kernel,
        mesh=mesh,
        out_type=jax.ShapeDtypeStruct((npad, c), x.dtype),
        scratch_types=[
            pltpu.VMEM((nch, k_top, tch), jnp.int32),
            pltpu.VMEM((tch, c), x.dtype),
            pltpu.SemaphoreType.DMA,
        ],
    )
    def run(x_hbm, p_hbm, xs_hbm, idx_v, rows_v, ssem):
        wid = lax.axis_index("s") * info.num_cores + lax.axis_index("c")
        base = wid * tpw
        pltpu.sync_copy(p_hbm.at[wid], idx_v)
        for ch in range(nch):
            pltpu.sync_copy(x_hbm.at[pl.ds(base + ch * tch, tch)], rows_v)
            copies = [
                pltpu.async_copy(rows_v, xs_hbm.at[idx_v.at[ch, kk]], ssem)
                for kk in range(k_top)
            ]
            for cp in copies:
                cp.wait()

    return run(x, p)


def _sc_combine(ys, pos, n_rows):
    c = ys.shape[1]
    info = plsc.get_sparse_core_info()
    nw = info.num_cores * info.num_subcores
    rpw = n_rows // nw
    ch_sz = 64
    nch = rpw // ch_sz
    p = pos.reshape(nw, nch, ch_sz)
    mesh = plsc.VectorSubcoreMesh(core_axis_name="c", subcore_axis_name="s")

    @functools.partial(
        pl.kernel,
        mesh=mesh,
        out_type=jax.ShapeDtypeStruct((n_rows, c), jnp.float32),
        scratch_types=[
            pltpu.VMEM((nch, ch_sz), jnp.int32),
            pltpu.VMEM((ch_sz, c), jnp.float32),
            pltpu.SemaphoreType.DMA,
        ],
    )
    def run(ys_hbm, p_hbm, out_hbm, idx_v, rows_v, gsem):
        wid = lax.axis_index("s") * info.num_cores + lax.axis_index("c")
        base = wid * rpw
        pltpu.sync_copy(p_hbm.at[wid], idx_v)
        for ch in range(nch):
            pltpu.async_copy(ys_hbm.at[idx_v.at[ch]], rows_v, gsem).wait()
            pltpu.sync_copy(rows_v, out_hbm.at[pl.ds(base + ch * ch_sz, ch_sz)])

    return run(ys, p)


def kernel(x, idx, w, b):
    n_tok, in_c = x.shape
    k_top = idx.shape[1]
    n_exp = w.shape[0]
    out_c = w.shape[2]
    s = n_tok * k_top

    e = idx.reshape(s).astype(jnp.int32)
    oh = (e[:, None] == jnp.arange(n_exp, dtype=jnp.int32)[None, :]).astype(jnp.int32)
    csum = jnp.cumsum(oh, axis=0)
    cnt = csum[-1]
    rank = jnp.sum(csum * oh, axis=1) - 1
    blocks_e = (cnt + BLK - 1) // BLK
    cumblk = jnp.cumsum(blocks_e)
    nblocks = s // BLK + n_exp - 1
    start_row = (cumblk - blocks_e) * BLK
    pos = rank + jnp.sum(oh * start_row[None, :], axis=1)
    block_expert = jnp.minimum(
        jnp.sum(
            jnp.arange(nblocks, dtype=jnp.int32)[:, None] >= cumblk[None, :], axis=1
        ),
        n_exp - 1,
    ).astype(jnp.int32)

    pos2 = pos.reshape(n_tok, k_top)
    nb_used = cumblk[-1:].astype(jnp.int32)
    xs = _sc_dispatch(x, [pos2[:, kk] for kk in range(k_top)], nblocks * BLK)
    ys = _stacked_mm(xs, w, b, block_expert, nb_used, nblocks)
    out_flat = _sc_combine(ys, pos, s)
    return out_flat.reshape(n_tok, k_top, out_c)
